# baseline (speedup 1.0000x reference)
import jax
import jax.numpy as jnp
from jax import lax
from jax.experimental import pallas as pl
from jax.experimental.pallas import tpu as pltpu
from jax.experimental.pallas import tpu_sc as plsc

N = 100000
M = 1000
D = 128

BLK = 1800
NBLK = 24
K1 = BLK * NBLK

N2 = N - K1
CH = 160
NC, NS = 2, 16
NW = NC * NS
FULL = N2 // CH
TPW = (FULL + NW - 1) // NW
NB = 3
LA = NB - 1


def _tc_body(x_ref, vT_ref, idx_ref, out_ref):
    idxr = idx_ref[0]
    hi = idxr >> 5
    lo = idxr & 31
    iota = lax.broadcasted_iota(jnp.int32, (32, BLK), 0)
    oh_hi = (iota == hi).astype(jnp.float32)
    rows_mat = jnp.dot(vT_ref[...], oh_hi, preferred_element_type=jnp.float32)
    col = jnp.sum(jnp.where(iota == lo, rows_mat, 0.0), axis=0, keepdims=True)
    out_ref[:, :D] = x_ref[...]
    out_ref[:, D:D + 1] = col.reshape(BLK, 1)


def _sc_body(x_hbm, vals_hbm, idx_hbm, out_hbm,
             buf0, buf1, buf2, ib0, ib1, ib2, tab,
             sx0, sx1, sx2, si0, si1, si2, so0, so1, so2):
    bufs = (buf0, buf1, buf2)
    ibs = (ib0, ib1, ib2)
    sxs = (sx0, sx1, sx2)
    sis = (si0, si1, si2)
    sos = (so0, so1, so2)
    wid = lax.axis_index("s") * NC + lax.axis_index("c")
    pltpu.sync_copy(vals_hbm, tab)
    col128 = jnp.full((16,), D, jnp.int32)
    riota = lax.broadcasted_iota(jnp.int32, (16,), 0)

    def in_copies(t):
        cid = wid + NW * t
        s = t % NB
        return (
            pltpu.make_async_copy(
                idx_hbm.at[pl.ds(K1 + cid * CH, CH)], ibs[s], sis[s]),
            pltpu.make_async_copy(
                x_hbm.at[pl.ds(K1 + cid * CH, CH), :], bufs[s].at[:, 0:D], sxs[s]),
        )

    def out_copy(t):
        cid = wid + NW * t
        s = t % NB
        return pltpu.make_async_copy(
            bufs[s], out_hbm.at[pl.ds(cid * CH, CH), :], sos[s])

    def fill(t):
        s = t % NB
        for j in range(CH // 16):
            iv = ibs[s][pl.ds(j * 16, 16)]
            vals = plsc.load_gather(tab, [iv])
            plsc.store_scatter(bufs[s], [riota + j * 16, col128], vals)

    waited = set()

    def wait_out(t):
        if t < 0 or t in waited:
            return
        waited.add(t)

        @pl.when(wid + NW * t < FULL)
        def _():
            out_copy(t).wait()

    def start_in(u):
        if u >= TPW:
            return
        wait_out(u - NB)

        @pl.when(wid + NW * u < FULL)
        def _():
            c1, c2 = in_copies(u)
            c1.start()
            c2.start()

    for t in range(LA):
        start_in(t)
    for t in range(TPW):
        start_in(t + LA)

        @pl.when(wid + NW * t < FULL)
        def _proc():
            c1, c2 = in_copies(t)
            c1.wait()
            c2.wait()
            fill(t)
            out_copy(t).start()

    for t in range(TPW):
        wait_out(t)


def kernel(per_atom_property_tensor, per_molecule_values, atomic_subsystem_indices):
    vals_p = jnp.zeros((1024,), jnp.float32).at[:M].set(per_molecule_values)
    v2dT = vals_p.reshape(32, 32).T
    idx3 = atomic_subsystem_indices[:K1].reshape(NBLK, 1, BLK)

    top = pl.pallas_call(
        _tc_body,
        grid=(NBLK,),
        in_specs=[
            pl.BlockSpec((BLK, D), lambda i: (i, 0)),
            pl.BlockSpec((32, 32), lambda i: (0, 0)),
            pl.BlockSpec((1, 1, BLK), lambda i: (i, 0, 0)),
        ],
        out_specs=pl.BlockSpec((BLK, D + 1), lambda i: (i, 0)),
        out_shape=jax.ShapeDtypeStruct((K1, D + 1), jnp.float32),
    )(per_atom_property_tensor, v2dT, idx3)

    mesh = plsc.VectorSubcoreMesh(
        core_axis_name="c", subcore_axis_name="s", num_cores=NC, num_subcores=NS)
    bottom = pl.kernel(
        _sc_body,
        out_type=jax.ShapeDtypeStruct((N2, D + 1), jnp.float32),
        mesh=mesh,
        scratch_types=(
            [pltpu.VMEM((CH, D + 1), jnp.float32)] * NB
            + [pltpu.VMEM((CH,), jnp.int32)] * NB
            + [pltpu.VMEM((1024,), jnp.float32)]
            + [pltpu.SemaphoreType.DMA] * (3 * NB)
        ),
        compiler_params=pltpu.CompilerParams(needs_layout_passes=False),
    )(per_atom_property_tensor, vals_p, atomic_subsystem_indices)

    out = jnp.empty((N, D + 1), jnp.float32)
    out = lax.dynamic_update_slice(out, top, (0, 0))
    out = lax.dynamic_update_slice(out, bottom, (K1, 0))
    return out


# speedup vs baseline: 3.5852x; 3.5852x over previous
import jax
import jax.numpy as jnp
from jax import lax
from jax.experimental import pallas as pl
from jax.experimental.pallas import tpu as pltpu
from jax.experimental.pallas import tpu_sc as plsc

N = 100000
M = 1000
D = 128
CH = 160
NC, NS = 2, 16
NW = NC * NS
FULL = N // CH
TPW = (FULL + NW - 1) // NW
NB = 3
LA = NB - 1


def _sc_body(x_hbm, vals_hbm, idx_hbm, out_hbm,
             buf0, buf1, buf2, iball, tab,
             sx0, sx1, sx2, so0, so1, so2):
    bufs = (buf0, buf1, buf2)
    sxs = (sx0, sx1, sx2)
    sos = (so0, so1, so2)
    wid = lax.axis_index("s") * NC + lax.axis_index("c")
    start = (wid * FULL) // NW
    count = ((wid + 1) * FULL) // NW - start
    pltpu.sync_copy(vals_hbm, tab)
    pltpu.sync_copy(idx_hbm.at[pl.ds(start * CH, TPW * CH)], iball)
    col128 = jnp.full((16,), D, jnp.int32)
    riota = lax.broadcasted_iota(jnp.int32, (16,), 0)

    def in_copy(t):
        s = t % NB
        return pltpu.make_async_copy(
            x_hbm.at[pl.ds((start + t) * CH, CH), :], bufs[s].at[:, 0:D], sxs[s])

    def out_copy(t):
        s = t % NB
        return pltpu.make_async_copy(
            bufs[s], out_hbm.at[pl.ds((start + t) * CH, CH), :], sos[s])

    def fill(t):
        s = t % NB
        for j in range(CH // 16):
            iv = iball[pl.ds(t * CH + j * 16, 16)]
            vals = plsc.load_gather(tab, [iv])
            plsc.store_scatter(bufs[s], [riota + j * 16, col128], vals)

    waited = set()

    def wait_out(t):
        if t < 0 or t in waited:
            return
        waited.add(t)

        @pl.when(t < count)
        def _():
            out_copy(t).wait()

    def start_in(u):
        if u >= TPW:
            return
        wait_out(u - NB)

        @pl.when(u < count)
        def _():
            in_copy(u).start()

    for t in range(LA):
        start_in(t)
    for t in range(TPW):
        start_in(t + LA)

        @pl.when(t < count)
        def _proc():
            in_copy(t).wait()
            fill(t)
            out_copy(t).start()

    for t in range(TPW):
        wait_out(t)


def kernel(per_atom_property_tensor, per_molecule_values, atomic_subsystem_indices):
    vals_p = jnp.zeros((1024,), jnp.float32).at[:M].set(per_molecule_values)
    mesh = plsc.VectorSubcoreMesh(
        core_axis_name="c", subcore_axis_name="s", num_cores=NC, num_subcores=NS)
    f = pl.kernel(
        _sc_body,
        out_type=jax.ShapeDtypeStruct((N, D + 1), jnp.float32),
        mesh=mesh,
        scratch_types=(
            [pltpu.VMEM((CH, D + 1), jnp.float32)] * NB
            + [pltpu.VMEM((TPW * CH,), jnp.int32)]
            + [pltpu.VMEM((1024,), jnp.float32)]
            + [pltpu.SemaphoreType.DMA] * (2 * NB)
        ),
        compiler_params=pltpu.CompilerParams(
            needs_layout_passes=False),
    )
    return f(per_atom_property_tensor, vals_p, atomic_subsystem_indices)


# speedup vs baseline: 3.6003x; 1.0042x over previous
import jax
import jax.numpy as jnp
from jax import lax
from jax.experimental import pallas as pl
from jax.experimental.pallas import tpu as pltpu
from jax.experimental.pallas import tpu_sc as plsc

N = 100000
M = 1000
D = 128
CH = 160
NC, NS = 2, 16
NW = NC * NS
FULL = N // CH
TPW = (FULL + NW - 1) // NW
NB = 3
LA = NB - 1


def _sc_body(x_hbm, vals_hbm, idx_hbm, out_hbm,
             buf0, buf1, buf2, iball, tab,
             sx0, sx1, sx2, so0, so1, so2):
    bufs = (buf0, buf1, buf2)
    sxs = (sx0, sx1, sx2)
    sos = (so0, so1, so2)
    wid = lax.axis_index("s") * NC + lax.axis_index("c")
    start = (wid * FULL) // NW
    count = ((wid + 1) * FULL) // NW - start
    pltpu.sync_copy(vals_hbm, tab)
    pltpu.sync_copy(idx_hbm.at[pl.ds(start * CH, TPW * CH)], iball)
    col128 = jnp.full((16,), D, jnp.int32)
    riota = lax.broadcasted_iota(jnp.int32, (16,), 0)

    def in_copy(t):
        s = t % NB
        return pltpu.make_async_copy(
            x_hbm.at[pl.ds((start + t) * CH, CH), :], bufs[s].at[:, 0:D], sxs[s])

    def out_copy(t):
        s = t % NB
        return pltpu.make_async_copy(
            bufs[s], out_hbm.at[pl.ds((start + t) * CH, CH), :], sos[s])

    def fill(t):
        s = t % NB
        for j in range(CH // 16):
            iv = iball[pl.ds(t * CH + j * 16, 16)]
            vals = plsc.load_gather(tab, [iv])
            plsc.store_scatter(bufs[s], [riota + j * 16, col128], vals)

    waited = set()

    def wait_out(t):
        if t < 0 or t in waited:
            return
        waited.add(t)

        @pl.when(t < count)
        def _():
            out_copy(t).wait()

    def start_in(u):
        if u >= TPW:
            return
        wait_out(u - NB)

        @pl.when(u < count)
        def _():
            in_copy(u).start()

    for t in range(LA):
        start_in(t)
    for t in range(TPW):
        start_in(t + LA)

        @pl.when(t < count)
        def _proc():
            fill(t)
            in_copy(t).wait()
            out_copy(t).start()

    for t in range(TPW):
        wait_out(t)


def kernel(per_atom_property_tensor, per_molecule_values, atomic_subsystem_indices):
    vals_p = jnp.zeros((1024,), jnp.float32).at[:M].set(per_molecule_values)
    mesh = plsc.VectorSubcoreMesh(
        core_axis_name="c", subcore_axis_name="s", num_cores=NC, num_subcores=NS)
    f = pl.kernel(
        _sc_body,
        out_type=jax.ShapeDtypeStruct((N, D + 1), jnp.float32),
        mesh=mesh,
        scratch_types=(
            [pltpu.VMEM((CH, D + 1), jnp.float32)] * NB
            + [pltpu.VMEM((TPW * CH,), jnp.int32)]
            + [pltpu.VMEM((1024,), jnp.float32)]
            + [pltpu.SemaphoreType.DMA] * (2 * NB)
        ),
        compiler_params=pltpu.CompilerParams(
            needs_layout_passes=False),
    )
    return f(per_atom_property_tensor, vals_p, atomic_subsystem_indices)
